# baseline (speedup 1.0000x reference)
import functools

import jax
import jax.numpy as jnp
import numpy as np
from jax import lax
from jax.experimental import pallas as pl
from jax.experimental.pallas import tpu as pltpu
from jax.experimental.pallas import tpu_sc as plsc

N = 10000
E = 160000
C = 151
R = 51
EMB = 128
D_IN = 512
H = 256
HH = 128
H_OUT = 16
N_LAYERS = 2

FQ = 64
NQ = 4
NA = 10240
NPA = 23040
EPAD = 163840
TCH = 160
KCH = 64
PW = EPAD // 32
PCH = PW // 16
BN = 1280

_mesh = plsc.VectorSubcoreMesh(core_axis_name="c", subcore_axis_name="s")

_TAU = np.zeros(H, np.int32)
for _k in range(NQ):
    for _g in range(2):
        for _i in range(16):
            _w = 32 * _k + 16 * _g + _i
            _TAU[_w] = FQ * _k + 32 * _g + _i
            _TAU[128 + _w] = FQ * _k + 32 * _g + 16 + _i
FQC = FQ // 2


@functools.partial(
    pl.kernel,
    out_type=[
        jax.ShapeDtypeStruct((32, PW), jnp.int32),
        jax.ShapeDtypeStruct((32, NA), jnp.float32),
    ],
    mesh=_mesh,
    scratch_types=[
        pltpu.VMEM((NA,), jnp.int32),
        pltpu.VMEM((PW,), jnp.int32),
        pltpu.VMEM((PW,), jnp.int32),
        pltpu.VMEM((PW,), jnp.int32),
        pltpu.VMEM((NA,), jnp.float32),
    ],
    compiler_params=pltpu.CompilerParams(needs_layout_passes=False),
)
def _pair_deg_kernel(labels_hbm, src_hbm, dst_hbm, pair_out, deg_out,
                     labels_v, src_v, dst_v, pair_v, deg_v):
    c = lax.axis_index("c")
    s = lax.axis_index("s")
    w = s * 2 + c
    pltpu.sync_copy(labels_hbm, labels_v)
    pltpu.sync_copy(src_hbm.at[w], src_v)
    pltpu.sync_copy(dst_hbm.at[w], dst_v)
    zeros16 = jnp.zeros((16,), jnp.float32)

    def zero_body(i, _):
        deg_v[pl.ds(i * 16, 16)] = zeros16
        return 0

    lax.fori_loop(0, NA // 16, zero_body, 0)
    ones16 = jnp.ones((16,), jnp.float32)

    def body(j, _):
        sv = src_v[pl.ds(j * 16, 16)]
        dv = dst_v[pl.ds(j * 16, 16)]
        ls = plsc.load_gather(labels_v, [sv])
        ld = plsc.load_gather(labels_v, [dv])
        pair_v[pl.ds(j * 16, 16)] = ls * C + ld
        plsc.addupdate_scatter(deg_v, [dv], ones16)
        return 0

    lax.fori_loop(0, PCH, body, 0)
    pltpu.sync_copy(pair_v, pair_out.at[w])
    pltpu.sync_copy(deg_v, deg_out.at[w])


def _make_message_kernel(q):
    @functools.partial(
        pl.kernel,
        out_type=jax.ShapeDtypeStruct((2 * NA, FQ), jnp.float32),
        mesh=_mesh,
        scratch_types=[
            pltpu.VMEM((TCH, KCH), jnp.int32),
            pltpu.VMEM((TCH, KCH), jnp.int32),
            pltpu.VMEM((TCH, KCH), jnp.int32),
            pltpu.VMEM((4, KCH, FQ), jnp.float32),
            pltpu.VMEM((4, KCH, FQC), jnp.float32),
            pltpu.VMEM((KCH, FQ), jnp.float32),
            pltpu.VMEM_SHARED((NA, FQ), jnp.float32),
        ] + [pltpu.SemaphoreType.DMA] * 12,
        compiler_params=pltpu.CompilerParams(needs_layout_passes=False,
                                             use_tc_tiling_on_sc=False),
    )
    def _message_kernel(rep_hbm, gate_hbm, src_hbm, pair_hbm, dst_hbm,
                        agg_out, src_v, pair_v, dst_v, rep_b, gate_b, zero_b,
                        acc, *sems):
        c = lax.axis_index("c")
        s = lax.axis_index("s")
        pltpu.sync_copy(src_hbm.at[s], src_v)
        pltpu.sync_copy(pair_hbm.at[s], pair_v)
        pltpu.sync_copy(dst_hbm.at[s], dst_v)
        koff = 2 * q + c
        c_rep = koff * NA
        c_gate = koff * NPA

        def off_body(j, _):
            for i in range(KCH // 16):
                sl = pl.ds(i * 16, 16)
                src_v[j, sl] = src_v[j, sl] + c_rep
                pair_v[j, sl] = pair_v[j, sl] + c_gate
            return 0

        lax.fori_loop(0, TCH, off_body, 0)

        zeros16 = jnp.zeros((16,), jnp.float32)

        def zb_body(r, _):
            for l in range(FQ // 16):
                zero_b[r, pl.ds(l * 16, 16)] = zeros16
            return 0

        lax.fori_loop(0, KCH, zb_body, 0)

        rows_per_tile = NA // 16

        def zacc_body(k, _):
            pltpu.sync_copy(zero_b,
                            acc.at[pl.ds(s * rows_per_tile + k * KCH, KCH)])
            return 0

        lax.fori_loop(0, rows_per_tile // KCH, zacc_body, 0)
        plsc.subcore_barrier()

        semr = sems[0:4]
        semg = sems[4:8]
        sems_sc = sems[8:12]
        RPG = 8

        def issue_gather(j, b):
            pltpu.async_copy(rep_hbm.at[src_v.at[j]], rep_b.at[b], semr[b])
            pltpu.async_copy(gate_hbm.at[pair_v.at[j]], gate_b.at[b], semg[b])

        def wait_gather(j, b):
            pltpu.make_async_copy(rep_hbm.at[src_v.at[j]], rep_b.at[b],
                                  semr[b]).wait()
            pltpu.make_async_copy(gate_hbm.at[pair_v.at[j]], gate_b.at[b],
                                  semg[b]).wait()

        def issue_scatter(j, b):
            pltpu.async_copy(rep_b.at[b], acc.at[dst_v.at[j]], sems_sc[b],
                             add=True)

        def wait_scatter(j, b):
            pltpu.make_async_copy(rep_b.at[b], acc.at[dst_v.at[j]],
                                  sems_sc[b]).wait()

        def mul(b):
            def mul_body(g, _):
                for rr in range(RPG):
                    r = g * RPG + rr
                    for g2 in range(2):
                        w16 = gate_b[b, r, pl.ds(16 * g2, 16)]
                        gpair = plsc.bitcast(w16, jnp.bfloat16)
                        ga, gb = plsc.unpack(
                            gpair, format=plsc.PackFormat.INTERLEAVED,
                            preferred_element_type=jnp.float32)
                        sla = pl.ds(32 * g2, 16)
                        slb = pl.ds(32 * g2 + 16, 16)
                        rep_b[b, r, sla] = rep_b[b, r, sla] * ga
                        rep_b[b, r, slb] = rep_b[b, r, slb] * gb
                return 0

            lax.fori_loop(0, KCH // RPG, mul_body, 0)

        issue_gather(0, 0)
        issue_gather(1, 1)
        issue_gather(2, 2)
        wait_gather(0, 0)
        mul(0)
        issue_scatter(0, 0)
        issue_gather(3, 3)

        def edge_body(jj, _):
            for i in range(4):
                j = jj * 4 + i + 1
                b = (i + 1) % 4
                bp = (b + 3) % 4
                wait_scatter(j - 1, bp)
                issue_gather(j + 3, bp)
                wait_gather(j, b)
                mul(b)
                issue_scatter(j, b)
            return 0

        lax.fori_loop(0, (TCH - 4) // 4, edge_body, 0)
        for j in (TCH - 3, TCH - 2, TCH - 1):
            b = j % 4
            wait_gather(j, b)
            mul(b)
            issue_scatter(j, b)
        for j in (TCH - 4, TCH - 3, TCH - 2, TCH - 1):
            wait_scatter(j, j % 4)
        plsc.subcore_barrier()

        def out_body(k, _):
            sl = pl.ds(s * rows_per_tile + k * KCH, KCH)
            osl = pl.ds(c * NA + s * rows_per_tile + k * KCH, KCH)
            pltpu.sync_copy(acc.at[sl], agg_out.at[osl])
            return 0

        lax.fori_loop(0, rows_per_tile // KCH, out_body, 0)

    return _message_kernel


_message_kernels = [_make_message_kernel(0), _make_message_kernel(1)]


def _pos_mlp_body(pf_ref, w1_ref, b1_ref, w2_ref, b2_ref, out_ref):
    h1 = jnp.dot(pf_ref[...], w1_ref[...], preferred_element_type=jnp.float32)
    h1 = h1 + b1_ref[...]
    mu = jnp.mean(h1, axis=0, keepdims=True)
    var = jnp.mean((h1 - mu) * (h1 - mu), axis=0, keepdims=True)
    h1 = (h1 - mu) / jnp.sqrt(var + 1e-5)
    h2 = jnp.dot(h1, w2_ref[...], preferred_element_type=jnp.float32)
    out_ref[...] = jnp.maximum(h2 + b2_ref[...], 0.0)


def _pos_mlp(pos_feats, W_pos1, b_pos1, W_pos2, b_pos2):
    return pl.pallas_call(
        _pos_mlp_body,
        out_shape=jax.ShapeDtypeStruct((N, 128), jnp.float32),
    )(pos_feats, W_pos1, b_pos1.reshape(1, 32), W_pos2, b_pos2.reshape(1, 128))


def _encoder_body(x_ref, lab_ref, pos_ref, wx_ref, we_ref, wp_ref, b_ref,
                  emb_ref, out_ref):
    lab = lab_ref[...]
    iota = lax.broadcasted_iota(jnp.int32, (BN, C), 1)
    onehot = (lab == iota).astype(jnp.float32)
    emb = jnp.dot(onehot, emb_ref[...], preferred_element_type=jnp.float32)
    rep = (jnp.dot(x_ref[...], wx_ref[...], preferred_element_type=jnp.float32)
           + jnp.dot(emb, we_ref[...], preferred_element_type=jnp.float32)
           + jnp.dot(pos_ref[...], wp_ref[...], preferred_element_type=jnp.float32)
           + b_ref[...])
    for j in range(NQ):
        out_ref[j] = rep[:, FQ * j:FQ * (j + 1)]


def _encoder(x_pad, labels2, pos_pad, W_enc, b_enc, obj_embed_w):
    wx = W_enc[:D_IN]
    we = W_enc[D_IN:D_IN + EMB]
    wp = W_enc[D_IN + EMB:]
    grid = NA // BN
    return pl.pallas_call(
        _encoder_body,
        grid=(grid,),
        in_specs=[
            pl.BlockSpec((BN, D_IN), lambda i: (i, 0)),
            pl.BlockSpec((BN, 1), lambda i: (i, 0)),
            pl.BlockSpec((BN, 128), lambda i: (i, 0)),
            pl.BlockSpec((D_IN, H), lambda i: (0, 0)),
            pl.BlockSpec((EMB, H), lambda i: (0, 0)),
            pl.BlockSpec((128, H), lambda i: (0, 0)),
            pl.BlockSpec((1, H), lambda i: (0, 0)),
            pl.BlockSpec((C, EMB), lambda i: (0, 0)),
        ],
        out_specs=pl.BlockSpec((NQ, BN, FQ), lambda i: (0, i, 0)),
        out_shape=jax.ShapeDtypeStruct((NQ, NA, FQ), jnp.float32),
    )(x_pad, labels2, pos_pad, wx, we, wp, b_enc.reshape(1, H), obj_embed_w)


def _gate_body(fb_ref, w_ref, b_ref, out_ref):
    g = jnp.dot(fb_ref[...], w_ref[...], preferred_element_type=jnp.float32)
    g = jax.nn.sigmoid(g + b_ref[...])
    au = jax.lax.bitcast_convert_type(g[:, :128], jnp.int32)
    bu = jax.lax.bitcast_convert_type(g[:, 128:], jnp.int32)
    ra = (au + 0x7FFF + ((au >> 16) & 1)) >> 16
    rb = (bu + 0x7FFF + ((bu >> 16) & 1)) >> 16
    cont = jax.lax.bitcast_convert_type((ra & 0xFFFF) | (rb << 16),
                                        jnp.float32)
    for j in range(NQ):
        out_ref[j] = cont[:, FQC * j:FQC * (j + 1)]


def _gate_table(fb_pad, W_gate_t, b_gate_t):
    grid = NPA // BN
    return pl.pallas_call(
        _gate_body,
        grid=(grid,),
        in_specs=[
            pl.BlockSpec((BN, R), lambda i: (i, 0)),
            pl.BlockSpec((R, H), lambda i: (0, 0)),
            pl.BlockSpec((1, H), lambda i: (0, 0)),
        ],
        out_specs=pl.BlockSpec((NQ, BN, FQC), lambda i: (0, i, 0)),
        out_shape=jax.ShapeDtypeStruct((NQ, NPA, FQC), jnp.float32),
    )(fb_pad, W_gate_t, b_gate_t.reshape(1, H))


def _assemble(rep_ref, agg0_ref, agg1_ref, deg_ref, ws_ref, wm_ref, b_ref):
    rep = jnp.concatenate([rep_ref[j] for j in range(NQ)], axis=1)
    agg = jnp.concatenate(
        [agg0_ref[0], agg0_ref[1], agg1_ref[0], agg1_ref[1]], axis=1)
    deg = jnp.maximum(jnp.sum(deg_ref[...], axis=0), 1.0)
    agg = agg / deg
    h = (jnp.dot(rep, ws_ref[...], preferred_element_type=jnp.float32)
         + jnp.dot(agg, wm_ref[...], preferred_element_type=jnp.float32)
         + b_ref[...])
    return jnp.maximum(h, 0.0)


def _update_mid_body(rep_ref, agg0_ref, agg1_ref, deg_ref, ws_ref, wm_ref,
                     b_ref, out_ref):
    h = _assemble(rep_ref, agg0_ref, agg1_ref, deg_ref, ws_ref, wm_ref, b_ref)
    for j in range(NQ):
        out_ref[j] = h[:, FQ * j:FQ * (j + 1)]


def _update_mid(rep4, agg0, agg1, degp3, W_upd_self, W_upd_msg, b_upd):
    grid = NA // BN
    return pl.pallas_call(
        _update_mid_body,
        grid=(grid,),
        in_specs=[
            pl.BlockSpec((NQ, BN, FQ), lambda i: (0, i, 0)),
            pl.BlockSpec((2, BN, FQ), lambda i: (0, i, 0)),
            pl.BlockSpec((2, BN, FQ), lambda i: (0, i, 0)),
            pl.BlockSpec((32, BN, 1), lambda i: (0, i, 0)),
            pl.BlockSpec((H, H), lambda i: (0, 0)),
            pl.BlockSpec((H, H), lambda i: (0, 0)),
            pl.BlockSpec((1, H), lambda i: (0, 0)),
        ],
        out_specs=pl.BlockSpec((NQ, BN, FQ), lambda i: (0, i, 0)),
        out_shape=jax.ShapeDtypeStruct((NQ, NA, FQ), jnp.float32),
    )(rep4, agg0, agg1, degp3, W_upd_self, W_upd_msg, b_upd.reshape(1, H))


def _update_final_body(rep_ref, agg0_ref, agg1_ref, deg_ref, ws_ref, wm_ref,
                       b_ref, wc_ref, bc_ref, out_ref):
    h = _assemble(rep_ref, agg0_ref, agg1_ref, deg_ref, ws_ref, wm_ref, b_ref)
    out_ref[...] = (jnp.dot(h, wc_ref[...], preferred_element_type=jnp.float32)
                    + bc_ref[...])


def _update_final(rep4, agg0, agg1, degp3, W_upd_self, W_upd_msg, b_upd,
                  W_cat, b_cat):
    grid = NA // BN
    nout = C + H_OUT
    return pl.pallas_call(
        _update_final_body,
        grid=(grid,),
        in_specs=[
            pl.BlockSpec((NQ, BN, FQ), lambda i: (0, i, 0)),
            pl.BlockSpec((2, BN, FQ), lambda i: (0, i, 0)),
            pl.BlockSpec((2, BN, FQ), lambda i: (0, i, 0)),
            pl.BlockSpec((32, BN, 1), lambda i: (0, i, 0)),
            pl.BlockSpec((H, H), lambda i: (0, 0)),
            pl.BlockSpec((H, H), lambda i: (0, 0)),
            pl.BlockSpec((1, H), lambda i: (0, 0)),
            pl.BlockSpec((H, nout), lambda i: (0, 0)),
            pl.BlockSpec((1, nout), lambda i: (0, 0)),
        ],
        out_specs=pl.BlockSpec((BN, nout), lambda i: (i, 0)),
        out_shape=jax.ShapeDtypeStruct((NA, nout), jnp.float32),
    )(rep4, agg0, agg1, degp3, W_upd_self, W_upd_msg, b_upd.reshape(1, H),
      W_cat, b_cat.reshape(1, nout))


def kernel(x, pos_feats, obj_labels, rel_pair_idx, freq_bias, obj_embed_w,
           W_pos1, b_pos1, W_pos2, b_pos2, W_enc, b_enc, W_gate, b_gate,
           W_upd_self, W_upd_msg, b_upd, W_out, b_out, W_h, b_h):
    x_pad = jnp.pad(x, ((0, NA - N), (0, 0)))
    labels_pad = jnp.pad(obj_labels, (0, NA - N))
    labels2 = labels_pad.reshape(NA, 1)
    fb_pad = jnp.pad(freq_bias, ((0, NPA - C * C), (0, 0)))
    src = rel_pair_idx[:, 0]
    dst = rel_pair_idx[:, 1]
    src_pad = jnp.pad(src, (0, EPAD - E))
    dst_pad = jnp.pad(dst, (0, EPAD - E), constant_values=N)
    src_w = src_pad.reshape(32, PW)
    dst_w = dst_pad.reshape(32, PW)
    src_t = src_pad.reshape(16, TCH, KCH)
    dst_t = dst_pad.reshape(16, TCH, KCH)

    pair_w, degp = _pair_deg_kernel(labels_pad, src_w, dst_w)
    pair_t = pair_w.reshape(16, TCH, KCH)
    degp3 = degp.reshape(32, NA, 1)

    pos = _pos_mlp(pos_feats, W_pos1, b_pos1, W_pos2, b_pos2)
    pos_pad = jnp.pad(pos, ((0, NA - N), (0, 0)))
    rep4 = _encoder(x_pad, labels2, pos_pad, W_enc, b_enc, obj_embed_w)
    tau = jnp.asarray(_TAU)
    gate4 = _gate_table(fb_pad, jnp.take(W_gate, tau, axis=1),
                        jnp.take(b_gate, tau)).reshape(NQ * NPA, FQC)

    W_cat = jnp.concatenate([W_out, W_h], axis=1)
    b_cat = jnp.concatenate([b_out, b_h], axis=0)
    for layer in range(N_LAYERS):
        rep_flat = rep4.reshape(NQ * NA, FQ)
        agg0 = _message_kernels[0](rep_flat, gate4, src_t, pair_t, dst_t)
        agg1 = _message_kernels[1](rep_flat, gate4, src_t, pair_t, dst_t)
        agg0 = agg0.reshape(2, NA, FQ)
        agg1 = agg1.reshape(2, NA, FQ)
        if layer < N_LAYERS - 1:
            rep4 = _update_mid(rep4, agg0, agg1, degp3, W_upd_self,
                               W_upd_msg, b_upd)
        else:
            out = _update_final(rep4, agg0, agg1, degp3, W_upd_self,
                                W_upd_msg, b_upd, W_cat, b_cat)
    return out[:N]


# speedup vs baseline: 1.0338x; 1.0338x over previous
import functools

import jax
import jax.numpy as jnp
from jax import lax
from jax.experimental import pallas as pl
from jax.experimental.pallas import tpu as pltpu
from jax.experimental.pallas import tpu_sc as plsc

N = 10000
E = 160000
C = 151
R = 51
EMB = 128
D_IN = 512
H = 256
HH = 128
H_OUT = 16
N_LAYERS = 2

FQ = 64
NQ = 4
NA = 10240
NPA = 23040
EPAD = 163840
TCH = 160
KCH = 64
PW = EPAD // 32
PCH = PW // 16
BN = 1280

_mesh = plsc.VectorSubcoreMesh(core_axis_name="c", subcore_axis_name="s")


@functools.partial(
    pl.kernel,
    out_type=[
        jax.ShapeDtypeStruct((32, PW), jnp.int32),
        jax.ShapeDtypeStruct((32, NA), jnp.float32),
    ],
    mesh=_mesh,
    scratch_types=[
        pltpu.VMEM((NA,), jnp.int32),
        pltpu.VMEM((PW,), jnp.int32),
        pltpu.VMEM((PW,), jnp.int32),
        pltpu.VMEM((PW,), jnp.int32),
        pltpu.VMEM((NA,), jnp.float32),
    ],
    compiler_params=pltpu.CompilerParams(needs_layout_passes=False),
)
def _pair_deg_kernel(labels_hbm, src_hbm, dst_hbm, pair_out, deg_out,
                     labels_v, src_v, dst_v, pair_v, deg_v):
    c = lax.axis_index("c")
    s = lax.axis_index("s")
    w = s * 2 + c
    pltpu.sync_copy(labels_hbm, labels_v)
    pltpu.sync_copy(src_hbm.at[w], src_v)
    pltpu.sync_copy(dst_hbm.at[w], dst_v)
    zeros16 = jnp.zeros((16,), jnp.float32)

    def zero_body(i, _):
        deg_v[pl.ds(i * 16, 16)] = zeros16
        return 0

    lax.fori_loop(0, NA // 16, zero_body, 0)
    ones16 = jnp.ones((16,), jnp.float32)

    def body(j, _):
        sv = src_v[pl.ds(j * 16, 16)]
        dv = dst_v[pl.ds(j * 16, 16)]
        ls = plsc.load_gather(labels_v, [sv])
        ld = plsc.load_gather(labels_v, [dv])
        pair_v[pl.ds(j * 16, 16)] = ls * C + ld
        plsc.addupdate_scatter(deg_v, [dv], ones16)
        return 0

    lax.fori_loop(0, PCH, body, 0)
    pltpu.sync_copy(pair_v, pair_out.at[w])
    pltpu.sync_copy(deg_v, deg_out.at[w])


def _make_message_kernel(q):
    @functools.partial(
        pl.kernel,
        out_type=jax.ShapeDtypeStruct((2 * NA, FQ), jnp.float32),
        mesh=_mesh,
        scratch_types=[
            pltpu.VMEM((TCH, KCH), jnp.int32),
            pltpu.VMEM((TCH, KCH), jnp.int32),
            pltpu.VMEM((TCH, KCH), jnp.int32),
            pltpu.VMEM((4, KCH, FQ), jnp.float32),
            pltpu.VMEM((4, KCH, FQ), jnp.float32),
            pltpu.VMEM((KCH, FQ), jnp.float32),
            pltpu.VMEM_SHARED((NA, FQ), jnp.float32),
        ] + [pltpu.SemaphoreType.DMA] * 12,
        compiler_params=pltpu.CompilerParams(needs_layout_passes=False,
                                             use_tc_tiling_on_sc=False),
    )
    def _message_kernel(rep_hbm, gate_hbm, src_hbm, pair_hbm, dst_hbm,
                        agg_out, src_v, pair_v, dst_v, rep_b, gate_b, zero_b,
                        acc, *sems):
        c = lax.axis_index("c")
        s = lax.axis_index("s")
        pltpu.sync_copy(src_hbm.at[s], src_v)
        pltpu.sync_copy(pair_hbm.at[s], pair_v)
        pltpu.sync_copy(dst_hbm.at[s], dst_v)
        koff = 2 * q + c
        c_rep = koff * NA
        c_gate = koff * NPA

        def off_body(j, _):
            for i in range(KCH // 16):
                sl = pl.ds(i * 16, 16)
                src_v[j, sl] = src_v[j, sl] + c_rep
                pair_v[j, sl] = pair_v[j, sl] + c_gate
            return 0

        lax.fori_loop(0, TCH, off_body, 0)

        zeros16 = jnp.zeros((16,), jnp.float32)

        def zb_body(r, _):
            for l in range(FQ // 16):
                zero_b[r, pl.ds(l * 16, 16)] = zeros16
            return 0

        lax.fori_loop(0, KCH, zb_body, 0)

        rows_per_tile = NA // 16

        def zacc_body(k, _):
            pltpu.sync_copy(zero_b,
                            acc.at[pl.ds(s * rows_per_tile + k * KCH, KCH)])
            return 0

        lax.fori_loop(0, rows_per_tile // KCH, zacc_body, 0)
        plsc.subcore_barrier()

        semr = sems[0:4]
        semg = sems[4:8]
        sems_sc = sems[8:12]
        RPG = 16

        def issue_gather(j, b):
            pltpu.async_copy(rep_hbm.at[src_v.at[j]], rep_b.at[b], semr[b])
            pltpu.async_copy(gate_hbm.at[pair_v.at[j]], gate_b.at[b], semg[b])

        def wait_gather(j, b):
            pltpu.make_async_copy(rep_hbm.at[src_v.at[j]], rep_b.at[b],
                                  semr[b]).wait()
            pltpu.make_async_copy(gate_hbm.at[pair_v.at[j]], gate_b.at[b],
                                  semg[b]).wait()

        def issue_scatter(j, b):
            pltpu.async_copy(rep_b.at[b], acc.at[dst_v.at[j]], sems_sc[b],
                             add=True)

        def wait_scatter(j, b):
            pltpu.make_async_copy(rep_b.at[b], acc.at[dst_v.at[j]],
                                  sems_sc[b]).wait()

        def mul(b):
            def mul_body(g, _):
                for rr in range(RPG):
                    for l in range(FQ // 16):
                        sl = pl.ds(l * 16, 16)
                        r = g * RPG + rr
                        rep_b[b, r, sl] = rep_b[b, r, sl] * gate_b[b, r, sl]
                return 0

            lax.fori_loop(0, KCH // RPG, mul_body, 0)

        issue_gather(0, 0)
        issue_gather(1, 1)
        issue_gather(2, 2)
        wait_gather(0, 0)
        mul(0)
        issue_scatter(0, 0)
        issue_gather(3, 3)

        def edge_body(jj, _):
            for i in range(4):
                j = jj * 4 + i + 1
                b = (i + 1) % 4
                bp = (b + 3) % 4
                wait_scatter(j - 1, bp)
                issue_gather(j + 3, bp)
                wait_gather(j, b)
                mul(b)
                issue_scatter(j, b)
            return 0

        lax.fori_loop(0, (TCH - 4) // 4, edge_body, 0)
        for j in (TCH - 3, TCH - 2, TCH - 1):
            b = j % 4
            wait_gather(j, b)
            mul(b)
            issue_scatter(j, b)
        for j in (TCH - 4, TCH - 3, TCH - 2, TCH - 1):
            wait_scatter(j, j % 4)
        plsc.subcore_barrier()

        def out_body(k, _):
            sl = pl.ds(s * rows_per_tile + k * KCH, KCH)
            osl = pl.ds(c * NA + s * rows_per_tile + k * KCH, KCH)
            pltpu.sync_copy(acc.at[sl], agg_out.at[osl])
            return 0

        lax.fori_loop(0, rows_per_tile // KCH, out_body, 0)

    return _message_kernel


_message_kernels = [_make_message_kernel(0), _make_message_kernel(1)]


def _pos_mlp_body(pf_ref, w1_ref, b1_ref, w2_ref, b2_ref, out_ref):
    h1 = jnp.dot(pf_ref[...], w1_ref[...], preferred_element_type=jnp.float32)
    h1 = h1 + b1_ref[...]
    mu = jnp.mean(h1, axis=0, keepdims=True)
    var = jnp.mean((h1 - mu) * (h1 - mu), axis=0, keepdims=True)
    h1 = (h1 - mu) / jnp.sqrt(var + 1e-5)
    h2 = jnp.dot(h1, w2_ref[...], preferred_element_type=jnp.float32)
    out_ref[...] = jnp.maximum(h2 + b2_ref[...], 0.0)


def _pos_mlp(pos_feats, W_pos1, b_pos1, W_pos2, b_pos2):
    return pl.pallas_call(
        _pos_mlp_body,
        out_shape=jax.ShapeDtypeStruct((N, 128), jnp.float32),
    )(pos_feats, W_pos1, b_pos1.reshape(1, 32), W_pos2, b_pos2.reshape(1, 128))


def _encoder_body(x_ref, lab_ref, pos_ref, wx_ref, we_ref, wp_ref, b_ref,
                  emb_ref, out_ref):
    lab = lab_ref[...]
    iota = lax.broadcasted_iota(jnp.int32, (BN, C), 1)
    onehot = (lab == iota).astype(jnp.float32)
    emb = jnp.dot(onehot, emb_ref[...], preferred_element_type=jnp.float32)
    rep = (jnp.dot(x_ref[...], wx_ref[...], preferred_element_type=jnp.float32)
           + jnp.dot(emb, we_ref[...], preferred_element_type=jnp.float32)
           + jnp.dot(pos_ref[...], wp_ref[...], preferred_element_type=jnp.float32)
           + b_ref[...])
    for j in range(NQ):
        out_ref[j] = rep[:, FQ * j:FQ * (j + 1)]


def _encoder(x_pad, labels2, pos_pad, W_enc, b_enc, obj_embed_w):
    wx = W_enc[:D_IN]
    we = W_enc[D_IN:D_IN + EMB]
    wp = W_enc[D_IN + EMB:]
    grid = NA // BN
    return pl.pallas_call(
        _encoder_body,
        grid=(grid,),
        in_specs=[
            pl.BlockSpec((BN, D_IN), lambda i: (i, 0)),
            pl.BlockSpec((BN, 1), lambda i: (i, 0)),
            pl.BlockSpec((BN, 128), lambda i: (i, 0)),
            pl.BlockSpec((D_IN, H), lambda i: (0, 0)),
            pl.BlockSpec((EMB, H), lambda i: (0, 0)),
            pl.BlockSpec((128, H), lambda i: (0, 0)),
            pl.BlockSpec((1, H), lambda i: (0, 0)),
            pl.BlockSpec((C, EMB), lambda i: (0, 0)),
        ],
        out_specs=pl.BlockSpec((NQ, BN, FQ), lambda i: (0, i, 0)),
        out_shape=jax.ShapeDtypeStruct((NQ, NA, FQ), jnp.float32),
    )(x_pad, labels2, pos_pad, wx, we, wp, b_enc.reshape(1, H), obj_embed_w)


def _gate_body(fb_ref, w_ref, b_ref, out_ref):
    g = jnp.dot(fb_ref[...], w_ref[...], preferred_element_type=jnp.float32)
    g = jax.nn.sigmoid(g + b_ref[...])
    for j in range(NQ):
        out_ref[j] = g[:, FQ * j:FQ * (j + 1)]


def _gate_table(fb_pad, W_gate, b_gate):
    grid = NPA // BN
    return pl.pallas_call(
        _gate_body,
        grid=(grid,),
        in_specs=[
            pl.BlockSpec((BN, R), lambda i: (i, 0)),
            pl.BlockSpec((R, H), lambda i: (0, 0)),
            pl.BlockSpec((1, H), lambda i: (0, 0)),
        ],
        out_specs=pl.BlockSpec((NQ, BN, FQ), lambda i: (0, i, 0)),
        out_shape=jax.ShapeDtypeStruct((NQ, NPA, FQ), jnp.float32),
    )(fb_pad, W_gate, b_gate.reshape(1, H))


def _assemble(rep_ref, agg0_ref, agg1_ref, deg_ref, ws_ref, wm_ref, b_ref):
    rep = jnp.concatenate([rep_ref[j] for j in range(NQ)], axis=1)
    agg = jnp.concatenate(
        [agg0_ref[0], agg0_ref[1], agg1_ref[0], agg1_ref[1]], axis=1)
    deg = jnp.maximum(jnp.sum(deg_ref[...], axis=0), 1.0)
    agg = agg / deg
    h = (jnp.dot(rep, ws_ref[...], preferred_element_type=jnp.float32)
         + jnp.dot(agg, wm_ref[...], preferred_element_type=jnp.float32)
         + b_ref[...])
    return jnp.maximum(h, 0.0)


def _update_mid_body(rep_ref, agg0_ref, agg1_ref, deg_ref, ws_ref, wm_ref,
                     b_ref, out_ref):
    h = _assemble(rep_ref, agg0_ref, agg1_ref, deg_ref, ws_ref, wm_ref, b_ref)
    for j in range(NQ):
        out_ref[j] = h[:, FQ * j:FQ * (j + 1)]


def _update_mid(rep4, agg0, agg1, degp3, W_upd_self, W_upd_msg, b_upd):
    grid = NA // BN
    return pl.pallas_call(
        _update_mid_body,
        grid=(grid,),
        in_specs=[
            pl.BlockSpec((NQ, BN, FQ), lambda i: (0, i, 0)),
            pl.BlockSpec((2, BN, FQ), lambda i: (0, i, 0)),
            pl.BlockSpec((2, BN, FQ), lambda i: (0, i, 0)),
            pl.BlockSpec((32, BN, 1), lambda i: (0, i, 0)),
            pl.BlockSpec((H, H), lambda i: (0, 0)),
            pl.BlockSpec((H, H), lambda i: (0, 0)),
            pl.BlockSpec((1, H), lambda i: (0, 0)),
        ],
        out_specs=pl.BlockSpec((NQ, BN, FQ), lambda i: (0, i, 0)),
        out_shape=jax.ShapeDtypeStruct((NQ, NA, FQ), jnp.float32),
    )(rep4, agg0, agg1, degp3, W_upd_self, W_upd_msg, b_upd.reshape(1, H))


def _update_final_body(rep_ref, agg0_ref, agg1_ref, deg_ref, ws_ref, wm_ref,
                       b_ref, wc_ref, bc_ref, out_ref):
    h = _assemble(rep_ref, agg0_ref, agg1_ref, deg_ref, ws_ref, wm_ref, b_ref)
    out_ref[...] = (jnp.dot(h, wc_ref[...], preferred_element_type=jnp.float32)
                    + bc_ref[...])


def _update_final(rep4, agg0, agg1, degp3, W_upd_self, W_upd_msg, b_upd,
                  W_cat, b_cat):
    grid = NA // BN
    nout = C + H_OUT
    return pl.pallas_call(
        _update_final_body,
        grid=(grid,),
        in_specs=[
            pl.BlockSpec((NQ, BN, FQ), lambda i: (0, i, 0)),
            pl.BlockSpec((2, BN, FQ), lambda i: (0, i, 0)),
            pl.BlockSpec((2, BN, FQ), lambda i: (0, i, 0)),
            pl.BlockSpec((32, BN, 1), lambda i: (0, i, 0)),
            pl.BlockSpec((H, H), lambda i: (0, 0)),
            pl.BlockSpec((H, H), lambda i: (0, 0)),
            pl.BlockSpec((1, H), lambda i: (0, 0)),
            pl.BlockSpec((H, nout), lambda i: (0, 0)),
            pl.BlockSpec((1, nout), lambda i: (0, 0)),
        ],
        out_specs=pl.BlockSpec((BN, nout), lambda i: (i, 0)),
        out_shape=jax.ShapeDtypeStruct((NA, nout), jnp.float32),
    )(rep4, agg0, agg1, degp3, W_upd_self, W_upd_msg, b_upd.reshape(1, H),
      W_cat, b_cat.reshape(1, nout))


def kernel(x, pos_feats, obj_labels, rel_pair_idx, freq_bias, obj_embed_w,
           W_pos1, b_pos1, W_pos2, b_pos2, W_enc, b_enc, W_gate, b_gate,
           W_upd_self, W_upd_msg, b_upd, W_out, b_out, W_h, b_h):
    x_pad = jnp.pad(x, ((0, NA - N), (0, 0)))
    labels_pad = jnp.pad(obj_labels, (0, NA - N))
    labels2 = labels_pad.reshape(NA, 1)
    fb_pad = jnp.pad(freq_bias, ((0, NPA - C * C), (0, 0)))
    src = rel_pair_idx[:, 0]
    dst = rel_pair_idx[:, 1]
    src_pad = jnp.pad(src, (0, EPAD - E))
    dst_pad = jnp.pad(dst, (0, EPAD - E), constant_values=N)
    src_w = src_pad.reshape(32, PW)
    dst_w = dst_pad.reshape(32, PW)
    src_t = src_pad.reshape(16, TCH, KCH)
    dst_t = dst_pad.reshape(16, TCH, KCH)

    pair_w, degp = _pair_deg_kernel(labels_pad, src_w, dst_w)
    pair_t = pair_w.reshape(16, TCH, KCH)
    degp3 = degp.reshape(32, NA, 1)

    pos = _pos_mlp(pos_feats, W_pos1, b_pos1, W_pos2, b_pos2)
    pos_pad = jnp.pad(pos, ((0, NA - N), (0, 0)))
    rep4 = _encoder(x_pad, labels2, pos_pad, W_enc, b_enc, obj_embed_w)
    gate4 = _gate_table(fb_pad, W_gate, b_gate).reshape(NQ * NPA, FQ)

    W_cat = jnp.concatenate([W_out, W_h], axis=1)
    b_cat = jnp.concatenate([b_out, b_h], axis=0)
    for layer in range(N_LAYERS):
        rep_flat = rep4.reshape(NQ * NA, FQ)
        agg0 = _message_kernels[0](rep_flat, gate4, src_t, pair_t, dst_t)
        agg1 = _message_kernels[1](rep_flat, gate4, src_t, pair_t, dst_t)
        agg0 = agg0.reshape(2, NA, FQ)
        agg1 = agg1.reshape(2, NA, FQ)
        if layer < N_LAYERS - 1:
            rep4 = _update_mid(rep4, agg0, agg1, degp3, W_upd_self,
                               W_upd_msg, b_upd)
        else:
            out = _update_final(rep4, agg0, agg1, degp3, W_upd_self,
                                W_upd_msg, b_upd, W_cat, b_cat)
    return out[:N]


# speedup vs baseline: 1.0378x; 1.0039x over previous
import functools

import jax
import jax.numpy as jnp
from jax import lax
from jax.experimental import pallas as pl
from jax.experimental.pallas import tpu as pltpu
from jax.experimental.pallas import tpu_sc as plsc

N = 10000
E = 160000
C = 151
R = 51
EMB = 128
D_IN = 512
H = 256
HH = 128
H_OUT = 16
N_LAYERS = 2

FQ = 64
NQ = 4
NA = 10240
NPA = 23040
EPAD = 163840
TCH = 160
KCH = 64
PW = EPAD // 32
PCH = PW // 16
BN = 1280

_mesh = plsc.VectorSubcoreMesh(core_axis_name="c", subcore_axis_name="s")


@functools.partial(
    pl.kernel,
    out_type=[
        jax.ShapeDtypeStruct((32, PW), jnp.int32),
        jax.ShapeDtypeStruct((32, NA), jnp.float32),
    ],
    mesh=_mesh,
    scratch_types=[
        pltpu.VMEM((NA,), jnp.int32),
        pltpu.VMEM((PW,), jnp.int32),
        pltpu.VMEM((PW,), jnp.int32),
        pltpu.VMEM((PW,), jnp.int32),
        pltpu.VMEM((NA,), jnp.float32),
    ],
    compiler_params=pltpu.CompilerParams(needs_layout_passes=False),
)
def _pair_deg_kernel(labels_hbm, src_hbm, dst_hbm, pair_out, deg_out,
                     labels_v, src_v, dst_v, pair_v, deg_v):
    c = lax.axis_index("c")
    s = lax.axis_index("s")
    w = s * 2 + c
    pltpu.sync_copy(labels_hbm, labels_v)
    pltpu.sync_copy(src_hbm.at[w], src_v)
    pltpu.sync_copy(dst_hbm.at[w], dst_v)
    zeros16 = jnp.zeros((16,), jnp.float32)

    def zero_body(i, _):
        deg_v[pl.ds(i * 16, 16)] = zeros16
        return 0

    lax.fori_loop(0, NA // 16, zero_body, 0)
    ones16 = jnp.ones((16,), jnp.float32)

    def body(j, _):
        sv = src_v[pl.ds(j * 16, 16)]
        dv = dst_v[pl.ds(j * 16, 16)]
        ls = plsc.load_gather(labels_v, [sv])
        ld = plsc.load_gather(labels_v, [dv])
        pair_v[pl.ds(j * 16, 16)] = ls * C + ld
        plsc.addupdate_scatter(deg_v, [dv], ones16)
        return 0

    lax.fori_loop(0, PCH, body, 0)
    pltpu.sync_copy(pair_v, pair_out.at[w])
    pltpu.sync_copy(deg_v, deg_out.at[w])


def _make_message_kernel(q):
    @functools.partial(
        pl.kernel,
        out_type=jax.ShapeDtypeStruct((2 * NA, FQ), jnp.float32),
        mesh=_mesh,
        scratch_types=[
            pltpu.VMEM((TCH, KCH), jnp.int32),
            pltpu.VMEM((TCH, KCH), jnp.int32),
            pltpu.VMEM((TCH, KCH), jnp.int32),
            pltpu.VMEM((5, KCH, FQ), jnp.float32),
            pltpu.VMEM((5, KCH, FQ), jnp.float32),
            pltpu.VMEM((KCH, FQ), jnp.float32),
            pltpu.VMEM_SHARED((NA, FQ), jnp.float32),
        ] + [pltpu.SemaphoreType.DMA] * 15,
        compiler_params=pltpu.CompilerParams(needs_layout_passes=False,
                                             use_tc_tiling_on_sc=False),
    )
    def _message_kernel(rep_hbm, gate_hbm, src_hbm, pair_hbm, dst_hbm,
                        agg_out, src_v, pair_v, dst_v, rep_b, gate_b, zero_b,
                        acc, *sems):
        c = lax.axis_index("c")
        s = lax.axis_index("s")
        pltpu.sync_copy(src_hbm.at[s], src_v)
        pltpu.sync_copy(pair_hbm.at[s], pair_v)
        pltpu.sync_copy(dst_hbm.at[s], dst_v)
        koff = 2 * q + c
        c_rep = koff * NA
        c_gate = koff * NPA

        def off_body(j, _):
            for i in range(KCH // 16):
                sl = pl.ds(i * 16, 16)
                src_v[j, sl] = src_v[j, sl] + c_rep
                pair_v[j, sl] = pair_v[j, sl] + c_gate
            return 0

        lax.fori_loop(0, TCH, off_body, 0)

        zeros16 = jnp.zeros((16,), jnp.float32)

        def zb_body(r, _):
            for l in range(FQ // 16):
                zero_b[r, pl.ds(l * 16, 16)] = zeros16
            return 0

        lax.fori_loop(0, KCH, zb_body, 0)

        rows_per_tile = NA // 16

        def zacc_body(k, _):
            pltpu.sync_copy(zero_b,
                            acc.at[pl.ds(s * rows_per_tile + k * KCH, KCH)])
            return 0

        lax.fori_loop(0, rows_per_tile // KCH, zacc_body, 0)
        plsc.subcore_barrier()

        NBUF = 5
        semr = sems[0:NBUF]
        semg = sems[NBUF:2 * NBUF]
        sems_sc = sems[2 * NBUF:3 * NBUF]
        RPG = 8

        def issue_gather(j, b):
            pltpu.async_copy(rep_hbm.at[src_v.at[j]], rep_b.at[b], semr[b])
            pltpu.async_copy(gate_hbm.at[pair_v.at[j]], gate_b.at[b], semg[b])

        def wait_gather(j, b):
            pltpu.make_async_copy(rep_hbm.at[src_v.at[j]], rep_b.at[b],
                                  semr[b]).wait()
            pltpu.make_async_copy(gate_hbm.at[pair_v.at[j]], gate_b.at[b],
                                  semg[b]).wait()

        def issue_scatter(j, b):
            pltpu.async_copy(rep_b.at[b], acc.at[dst_v.at[j]], sems_sc[b],
                             add=True)

        def wait_scatter(j, b):
            pltpu.make_async_copy(rep_b.at[b], acc.at[dst_v.at[j]],
                                  sems_sc[b]).wait()

        def mul(b):
            def mul_body(g, _):
                for rr in range(RPG):
                    for l in range(FQ // 16):
                        sl = pl.ds(l * 16, 16)
                        r = g * RPG + rr
                        rep_b[b, r, sl] = rep_b[b, r, sl] * gate_b[b, r, sl]
                return 0

            lax.fori_loop(0, KCH // RPG, mul_body, 0)

        for b in range(NBUF - 1):
            issue_gather(b, b)
        wait_gather(0, 0)
        mul(0)
        issue_scatter(0, 0)
        issue_gather(NBUF - 1, NBUF - 1)

        def edge_body(jj, _):
            for i in range(NBUF):
                j = jj * NBUF + i + 1
                b = (i + 1) % NBUF
                bp = (b + NBUF - 1) % NBUF
                wait_scatter(j - 1, bp)
                issue_gather(j + NBUF - 1, bp)
                wait_gather(j, b)
                mul(b)
                issue_scatter(j, b)
            return 0

        lax.fori_loop(0, (TCH - NBUF) // NBUF, edge_body, 0)
        for j in range(TCH - NBUF + 1, TCH):
            b = j % NBUF
            wait_gather(j, b)
            mul(b)
            issue_scatter(j, b)
        for j in range(TCH - NBUF, TCH):
            wait_scatter(j, j % NBUF)
        plsc.subcore_barrier()

        def out_body(k, _):
            sl = pl.ds(s * rows_per_tile + k * KCH, KCH)
            osl = pl.ds(c * NA + s * rows_per_tile + k * KCH, KCH)
            pltpu.sync_copy(acc.at[sl], agg_out.at[osl])
            return 0

        lax.fori_loop(0, rows_per_tile // KCH, out_body, 0)

    return _message_kernel


_message_kernels = [_make_message_kernel(0), _make_message_kernel(1)]


def _pos_mlp_body(pf_ref, w1_ref, b1_ref, w2_ref, b2_ref, out_ref):
    h1 = jnp.dot(pf_ref[...], w1_ref[...], preferred_element_type=jnp.float32)
    h1 = h1 + b1_ref[...]
    mu = jnp.mean(h1, axis=0, keepdims=True)
    var = jnp.mean((h1 - mu) * (h1 - mu), axis=0, keepdims=True)
    h1 = (h1 - mu) / jnp.sqrt(var + 1e-5)
    h2 = jnp.dot(h1, w2_ref[...], preferred_element_type=jnp.float32)
    out_ref[...] = jnp.maximum(h2 + b2_ref[...], 0.0)


def _pos_mlp(pos_feats, W_pos1, b_pos1, W_pos2, b_pos2):
    return pl.pallas_call(
        _pos_mlp_body,
        out_shape=jax.ShapeDtypeStruct((N, 128), jnp.float32),
    )(pos_feats, W_pos1, b_pos1.reshape(1, 32), W_pos2, b_pos2.reshape(1, 128))


def _encoder_body(x_ref, lab_ref, pos_ref, wx_ref, we_ref, wp_ref, b_ref,
                  emb_ref, out_ref):
    lab = lab_ref[...]
    iota = lax.broadcasted_iota(jnp.int32, (BN, C), 1)
    onehot = (lab == iota).astype(jnp.float32)
    emb = jnp.dot(onehot, emb_ref[...], preferred_element_type=jnp.float32)
    rep = (jnp.dot(x_ref[...], wx_ref[...], preferred_element_type=jnp.float32)
           + jnp.dot(emb, we_ref[...], preferred_element_type=jnp.float32)
           + jnp.dot(pos_ref[...], wp_ref[...], preferred_element_type=jnp.float32)
           + b_ref[...])
    for j in range(NQ):
        out_ref[j] = rep[:, FQ * j:FQ * (j + 1)]


def _encoder(x_pad, labels2, pos_pad, W_enc, b_enc, obj_embed_w):
    wx = W_enc[:D_IN]
    we = W_enc[D_IN:D_IN + EMB]
    wp = W_enc[D_IN + EMB:]
    grid = NA // BN
    return pl.pallas_call(
        _encoder_body,
        grid=(grid,),
        in_specs=[
            pl.BlockSpec((BN, D_IN), lambda i: (i, 0)),
            pl.BlockSpec((BN, 1), lambda i: (i, 0)),
            pl.BlockSpec((BN, 128), lambda i: (i, 0)),
            pl.BlockSpec((D_IN, H), lambda i: (0, 0)),
            pl.BlockSpec((EMB, H), lambda i: (0, 0)),
            pl.BlockSpec((128, H), lambda i: (0, 0)),
            pl.BlockSpec((1, H), lambda i: (0, 0)),
            pl.BlockSpec((C, EMB), lambda i: (0, 0)),
        ],
        out_specs=pl.BlockSpec((NQ, BN, FQ), lambda i: (0, i, 0)),
        out_shape=jax.ShapeDtypeStruct((NQ, NA, FQ), jnp.float32),
    )(x_pad, labels2, pos_pad, wx, we, wp, b_enc.reshape(1, H), obj_embed_w)


def _gate_body(fb_ref, w_ref, b_ref, out_ref):
    g = jnp.dot(fb_ref[...], w_ref[...], preferred_element_type=jnp.float32)
    g = jax.nn.sigmoid(g + b_ref[...])
    for j in range(NQ):
        out_ref[j] = g[:, FQ * j:FQ * (j + 1)]


def _gate_table(fb_pad, W_gate, b_gate):
    grid = NPA // BN
    return pl.pallas_call(
        _gate_body,
        grid=(grid,),
        in_specs=[
            pl.BlockSpec((BN, R), lambda i: (i, 0)),
            pl.BlockSpec((R, H), lambda i: (0, 0)),
            pl.BlockSpec((1, H), lambda i: (0, 0)),
        ],
        out_specs=pl.BlockSpec((NQ, BN, FQ), lambda i: (0, i, 0)),
        out_shape=jax.ShapeDtypeStruct((NQ, NPA, FQ), jnp.float32),
    )(fb_pad, W_gate, b_gate.reshape(1, H))


def _assemble(rep_ref, agg0_ref, agg1_ref, deg_ref, ws_ref, wm_ref, b_ref):
    rep = jnp.concatenate([rep_ref[j] for j in range(NQ)], axis=1)
    agg = jnp.concatenate(
        [agg0_ref[0], agg0_ref[1], agg1_ref[0], agg1_ref[1]], axis=1)
    deg = jnp.maximum(jnp.sum(deg_ref[...], axis=0), 1.0)
    agg = agg / deg
    h = (jnp.dot(rep, ws_ref[...], preferred_element_type=jnp.float32)
         + jnp.dot(agg, wm_ref[...], preferred_element_type=jnp.float32)
         + b_ref[...])
    return jnp.maximum(h, 0.0)


def _update_mid_body(rep_ref, agg0_ref, agg1_ref, deg_ref, ws_ref, wm_ref,
                     b_ref, out_ref):
    h = _assemble(rep_ref, agg0_ref, agg1_ref, deg_ref, ws_ref, wm_ref, b_ref)
    for j in range(NQ):
        out_ref[j] = h[:, FQ * j:FQ * (j + 1)]


def _update_mid(rep4, agg0, agg1, degp3, W_upd_self, W_upd_msg, b_upd):
    grid = NA // BN
    return pl.pallas_call(
        _update_mid_body,
        grid=(grid,),
        in_specs=[
            pl.BlockSpec((NQ, BN, FQ), lambda i: (0, i, 0)),
            pl.BlockSpec((2, BN, FQ), lambda i: (0, i, 0)),
            pl.BlockSpec((2, BN, FQ), lambda i: (0, i, 0)),
            pl.BlockSpec((32, BN, 1), lambda i: (0, i, 0)),
            pl.BlockSpec((H, H), lambda i: (0, 0)),
            pl.BlockSpec((H, H), lambda i: (0, 0)),
            pl.BlockSpec((1, H), lambda i: (0, 0)),
        ],
        out_specs=pl.BlockSpec((NQ, BN, FQ), lambda i: (0, i, 0)),
        out_shape=jax.ShapeDtypeStruct((NQ, NA, FQ), jnp.float32),
    )(rep4, agg0, agg1, degp3, W_upd_self, W_upd_msg, b_upd.reshape(1, H))


def _update_final_body(rep_ref, agg0_ref, agg1_ref, deg_ref, ws_ref, wm_ref,
                       b_ref, wc_ref, bc_ref, out_ref):
    h = _assemble(rep_ref, agg0_ref, agg1_ref, deg_ref, ws_ref, wm_ref, b_ref)
    out_ref[...] = (jnp.dot(h, wc_ref[...], preferred_element_type=jnp.float32)
                    + bc_ref[...])


def _update_final(rep4, agg0, agg1, degp3, W_upd_self, W_upd_msg, b_upd,
                  W_cat, b_cat):
    grid = NA // BN
    nout = C + H_OUT
    return pl.pallas_call(
        _update_final_body,
        grid=(grid,),
        in_specs=[
            pl.BlockSpec((NQ, BN, FQ), lambda i: (0, i, 0)),
            pl.BlockSpec((2, BN, FQ), lambda i: (0, i, 0)),
            pl.BlockSpec((2, BN, FQ), lambda i: (0, i, 0)),
            pl.BlockSpec((32, BN, 1), lambda i: (0, i, 0)),
            pl.BlockSpec((H, H), lambda i: (0, 0)),
            pl.BlockSpec((H, H), lambda i: (0, 0)),
            pl.BlockSpec((1, H), lambda i: (0, 0)),
            pl.BlockSpec((H, nout), lambda i: (0, 0)),
            pl.BlockSpec((1, nout), lambda i: (0, 0)),
        ],
        out_specs=pl.BlockSpec((BN, nout), lambda i: (i, 0)),
        out_shape=jax.ShapeDtypeStruct((NA, nout), jnp.float32),
    )(rep4, agg0, agg1, degp3, W_upd_self, W_upd_msg, b_upd.reshape(1, H),
      W_cat, b_cat.reshape(1, nout))


def kernel(x, pos_feats, obj_labels, rel_pair_idx, freq_bias, obj_embed_w,
           W_pos1, b_pos1, W_pos2, b_pos2, W_enc, b_enc, W_gate, b_gate,
           W_upd_self, W_upd_msg, b_upd, W_out, b_out, W_h, b_h):
    x_pad = jnp.pad(x, ((0, NA - N), (0, 0)))
    labels_pad = jnp.pad(obj_labels, (0, NA - N))
    labels2 = labels_pad.reshape(NA, 1)
    fb_pad = jnp.pad(freq_bias, ((0, NPA - C * C), (0, 0)))
    src = rel_pair_idx[:, 0]
    dst = rel_pair_idx[:, 1]
    src_pad = jnp.pad(src, (0, EPAD - E))
    dst_pad = jnp.pad(dst, (0, EPAD - E), constant_values=N)
    src_w = src_pad.reshape(32, PW)
    dst_w = dst_pad.reshape(32, PW)
    src_t = src_pad.reshape(16, TCH, KCH)
    dst_t = dst_pad.reshape(16, TCH, KCH)

    pair_w, degp = _pair_deg_kernel(labels_pad, src_w, dst_w)
    pair_t = pair_w.reshape(16, TCH, KCH)
    degp3 = degp.reshape(32, NA, 1)

    pos = _pos_mlp(pos_feats, W_pos1, b_pos1, W_pos2, b_pos2)
    pos_pad = jnp.pad(pos, ((0, NA - N), (0, 0)))
    rep4 = _encoder(x_pad, labels2, pos_pad, W_enc, b_enc, obj_embed_w)
    gate4 = _gate_table(fb_pad, W_gate, b_gate).reshape(NQ * NPA, FQ)

    W_cat = jnp.concatenate([W_out, W_h], axis=1)
    b_cat = jnp.concatenate([b_out, b_h], axis=0)
    for layer in range(N_LAYERS):
        rep_flat = rep4.reshape(NQ * NA, FQ)
        agg0 = _message_kernels[0](rep_flat, gate4, src_t, pair_t, dst_t)
        agg1 = _message_kernels[1](rep_flat, gate4, src_t, pair_t, dst_t)
        agg0 = agg0.reshape(2, NA, FQ)
        agg1 = agg1.reshape(2, NA, FQ)
        if layer < N_LAYERS - 1:
            rep4 = _update_mid(rep4, agg0, agg1, degp3, W_upd_self,
                               W_upd_msg, b_upd)
        else:
            out = _update_final(rep4, agg0, agg1, degp3, W_upd_self,
                                W_upd_msg, b_upd, W_cat, b_cat)
    return out[:N]


# speedup vs baseline: 1.0428x; 1.0048x over previous
import functools

import jax
import jax.numpy as jnp
from jax import lax
from jax.experimental import pallas as pl
from jax.experimental.pallas import tpu as pltpu
from jax.experimental.pallas import tpu_sc as plsc

N = 10000
E = 160000
C = 151
R = 51
EMB = 128
D_IN = 512
H = 256
HH = 128
H_OUT = 16
N_LAYERS = 2

FQ = 64
NQ = 4
NA = 10240
NPA = 23040
EPAD = 163840
TCH = 160
KCH = 64
PW = EPAD // 32
PCH = PW // 16
BN = 1280

_mesh = plsc.VectorSubcoreMesh(core_axis_name="c", subcore_axis_name="s")


@functools.partial(
    pl.kernel,
    out_type=[
        jax.ShapeDtypeStruct((32, PW), jnp.int32),
        jax.ShapeDtypeStruct((32, NA), jnp.float32),
    ],
    mesh=_mesh,
    scratch_types=[
        pltpu.VMEM((NA,), jnp.int32),
        pltpu.VMEM((PW,), jnp.int32),
        pltpu.VMEM((PW,), jnp.int32),
        pltpu.VMEM((PW,), jnp.int32),
        pltpu.VMEM((NA,), jnp.float32),
    ],
    compiler_params=pltpu.CompilerParams(needs_layout_passes=False),
)
def _pair_deg_kernel(labels_hbm, src_hbm, dst_hbm, pair_out, deg_out,
                     labels_v, src_v, dst_v, pair_v, deg_v):
    c = lax.axis_index("c")
    s = lax.axis_index("s")
    w = s * 2 + c
    pltpu.sync_copy(labels_hbm, labels_v)
    pltpu.sync_copy(src_hbm.at[w], src_v)
    pltpu.sync_copy(dst_hbm.at[w], dst_v)
    zeros16 = jnp.zeros((16,), jnp.float32)

    def zero_body(i, _):
        deg_v[pl.ds(i * 16, 16)] = zeros16
        return 0

    lax.fori_loop(0, NA // 16, zero_body, 0)
    ones16 = jnp.ones((16,), jnp.float32)

    def body(j, _):
        sv = src_v[pl.ds(j * 16, 16)]
        dv = dst_v[pl.ds(j * 16, 16)]
        ls = plsc.load_gather(labels_v, [sv])
        ld = plsc.load_gather(labels_v, [dv])
        pair_v[pl.ds(j * 16, 16)] = ls * C + ld
        plsc.addupdate_scatter(deg_v, [dv], ones16)
        return 0

    lax.fori_loop(0, PCH, body, 0)
    pltpu.sync_copy(pair_v, pair_out.at[w])
    pltpu.sync_copy(deg_v, deg_out.at[w])


def _make_message_kernel(q):
    @functools.partial(
        pl.kernel,
        out_type=jax.ShapeDtypeStruct((2 * NA, FQ), jnp.float32),
        mesh=_mesh,
        scratch_types=[
            pltpu.VMEM((TCH, KCH), jnp.int32),
            pltpu.VMEM((TCH, KCH), jnp.int32),
            pltpu.VMEM((TCH, KCH), jnp.int32),
            pltpu.VMEM((4, KCH, FQ), jnp.float32),
            pltpu.VMEM((4, KCH, FQ), jnp.float32),
            pltpu.VMEM((KCH, FQ), jnp.float32),
            pltpu.VMEM_SHARED((NA, FQ), jnp.float32),
        ] + [pltpu.SemaphoreType.DMA] * 12,
        compiler_params=pltpu.CompilerParams(needs_layout_passes=False,
                                             use_tc_tiling_on_sc=False),
    )
    def _message_kernel(rep_hbm, gate_hbm, src_hbm, pair_hbm, dst_hbm,
                        agg_out, src_v, pair_v, dst_v, rep_b, gate_b, zero_b,
                        acc, *sems):
        c = lax.axis_index("c")
        s = lax.axis_index("s")
        pltpu.sync_copy(src_hbm.at[s], src_v)
        pltpu.sync_copy(pair_hbm.at[s], pair_v)
        pltpu.sync_copy(dst_hbm.at[s], dst_v)
        koff = 2 * q + c
        c_rep = koff * NA
        c_gate = koff * NPA

        def off_body(j, _):
            for i in range(KCH // 16):
                sl = pl.ds(i * 16, 16)
                src_v[j, sl] = src_v[j, sl] + c_rep
                pair_v[j, sl] = pair_v[j, sl] + c_gate
            return 0

        lax.fori_loop(0, TCH, off_body, 0)

        zeros16 = jnp.zeros((16,), jnp.float32)

        def zb_body(r, _):
            for l in range(FQ // 16):
                zero_b[r, pl.ds(l * 16, 16)] = zeros16
            return 0

        lax.fori_loop(0, KCH, zb_body, 0)

        rows_per_tile = NA // 16

        def zacc_body(k, _):
            pltpu.sync_copy(zero_b,
                            acc.at[pl.ds(s * rows_per_tile + k * KCH, KCH)])
            return 0

        lax.fori_loop(0, rows_per_tile // KCH, zacc_body, 0)
        plsc.subcore_barrier()

        semr = sems[0:4]
        semg = sems[4:8]
        sems_sc = sems[8:12]
        RPG = 8

        def issue_gather(j, b):
            pltpu.async_copy(rep_hbm.at[src_v.at[j]], rep_b.at[b], semr[b])
            pltpu.async_copy(gate_hbm.at[pair_v.at[j]], gate_b.at[b], semg[b])

        def wait_gather(j, b):
            pltpu.make_async_copy(rep_hbm.at[src_v.at[j]], rep_b.at[b],
                                  semr[b]).wait()
            pltpu.make_async_copy(gate_hbm.at[pair_v.at[j]], gate_b.at[b],
                                  semg[b]).wait()

        def issue_scatter(j, b):
            pltpu.async_copy(rep_b.at[b], acc.at[dst_v.at[j]], sems_sc[b],
                             add=True)

        def wait_scatter(j, b):
            pltpu.make_async_copy(rep_b.at[b], acc.at[dst_v.at[j]],
                                  sems_sc[b]).wait()

        def mul(b):
            def mul_body(g, _):
                for rr in range(RPG):
                    for l in range(FQ // 16):
                        sl = pl.ds(l * 16, 16)
                        r = g * RPG + rr
                        rep_b[b, r, sl] = rep_b[b, r, sl] * gate_b[b, r, sl]
                return 0

            lax.fori_loop(0, KCH // RPG, mul_body, 0)

        issue_gather(0, 0)
        issue_gather(1, 1)
        issue_gather(2, 2)
        wait_gather(0, 0)
        mul(0)
        issue_scatter(0, 0)
        issue_gather(3, 3)

        def edge_body(jj, _):
            for i in range(4):
                j = jj * 4 + i + 1
                b = (i + 1) % 4
                bp = (b + 3) % 4
                wait_scatter(j - 1, bp)
                issue_gather(j + 3, bp)
                wait_gather(j, b)
                mul(b)
                issue_scatter(j, b)
            return 0

        lax.fori_loop(0, (TCH - 4) // 4, edge_body, 0)
        for j in (TCH - 3, TCH - 2, TCH - 1):
            b = j % 4
            wait_gather(j, b)
            mul(b)
            issue_scatter(j, b)
        for j in (TCH - 4, TCH - 3, TCH - 2, TCH - 1):
            wait_scatter(j, j % 4)
        plsc.subcore_barrier()

        def out_body(k, _):
            sl = pl.ds(s * rows_per_tile + k * KCH, KCH)
            osl = pl.ds(c * NA + s * rows_per_tile + k * KCH, KCH)
            pltpu.sync_copy(acc.at[sl], agg_out.at[osl])
            return 0

        lax.fori_loop(0, rows_per_tile // KCH, out_body, 0)

    return _message_kernel


_message_kernels = [_make_message_kernel(0), _make_message_kernel(1)]


def _pos_mlp_body(pf_ref, w1_ref, b1_ref, w2_ref, b2_ref, out_ref):
    h1 = jnp.dot(pf_ref[...], w1_ref[...], preferred_element_type=jnp.float32)
    h1 = h1 + b1_ref[...]
    mu = jnp.mean(h1, axis=0, keepdims=True)
    var = jnp.mean((h1 - mu) * (h1 - mu), axis=0, keepdims=True)
    h1 = (h1 - mu) / jnp.sqrt(var + 1e-5)
    h2 = jnp.dot(h1, w2_ref[...], preferred_element_type=jnp.float32)
    out_ref[...] = jnp.maximum(h2 + b2_ref[...], 0.0)


def _pos_mlp(pos_feats, W_pos1, b_pos1, W_pos2, b_pos2):
    return pl.pallas_call(
        _pos_mlp_body,
        out_shape=jax.ShapeDtypeStruct((N, 128), jnp.float32),
    )(pos_feats, W_pos1, b_pos1.reshape(1, 32), W_pos2, b_pos2.reshape(1, 128))


def _encoder_body(x_ref, lab_ref, pos_ref, wx_ref, we_ref, wp_ref, b_ref,
                  emb_ref, out_ref):
    lab = lab_ref[...]
    iota = lax.broadcasted_iota(jnp.int32, (BN, C), 1)
    onehot = (lab == iota).astype(jnp.float32)
    emb = jnp.dot(onehot, emb_ref[...], preferred_element_type=jnp.float32)
    rep = (jnp.dot(x_ref[...], wx_ref[...], preferred_element_type=jnp.float32)
           + jnp.dot(emb, we_ref[...], preferred_element_type=jnp.float32)
           + jnp.dot(pos_ref[...], wp_ref[...], preferred_element_type=jnp.float32)
           + b_ref[...])
    for j in range(NQ):
        out_ref[j] = rep[:, FQ * j:FQ * (j + 1)]


def _encoder(x_pad, labels2, pos_pad, W_enc, b_enc, obj_embed_w):
    wx = W_enc[:D_IN]
    we = W_enc[D_IN:D_IN + EMB]
    wp = W_enc[D_IN + EMB:]
    grid = NA // BN
    return pl.pallas_call(
        _encoder_body,
        grid=(grid,),
        in_specs=[
            pl.BlockSpec((BN, D_IN), lambda i: (i, 0)),
            pl.BlockSpec((BN, 1), lambda i: (i, 0)),
            pl.BlockSpec((BN, 128), lambda i: (i, 0)),
            pl.BlockSpec((D_IN, H), lambda i: (0, 0)),
            pl.BlockSpec((EMB, H), lambda i: (0, 0)),
            pl.BlockSpec((128, H), lambda i: (0, 0)),
            pl.BlockSpec((1, H), lambda i: (0, 0)),
            pl.BlockSpec((C, EMB), lambda i: (0, 0)),
        ],
        out_specs=pl.BlockSpec((NQ, BN, FQ), lambda i: (0, i, 0)),
        out_shape=jax.ShapeDtypeStruct((NQ, NA, FQ), jnp.float32),
    )(x_pad, labels2, pos_pad, wx, we, wp, b_enc.reshape(1, H), obj_embed_w)


def _gate_body(fb_ref, w_ref, b_ref, out_ref):
    g = jnp.dot(fb_ref[...], w_ref[...], preferred_element_type=jnp.float32)
    g = jax.nn.sigmoid(g + b_ref[...])
    for j in range(NQ):
        out_ref[j] = g[:, FQ * j:FQ * (j + 1)]


def _gate_table(fb_pad, W_gate, b_gate):
    grid = NPA // BN
    return pl.pallas_call(
        _gate_body,
        grid=(grid,),
        in_specs=[
            pl.BlockSpec((BN, R), lambda i: (i, 0)),
            pl.BlockSpec((R, H), lambda i: (0, 0)),
            pl.BlockSpec((1, H), lambda i: (0, 0)),
        ],
        out_specs=pl.BlockSpec((NQ, BN, FQ), lambda i: (0, i, 0)),
        out_shape=jax.ShapeDtypeStruct((NQ, NPA, FQ), jnp.float32),
    )(fb_pad, W_gate, b_gate.reshape(1, H))


def _assemble(rep_ref, agg0_ref, agg1_ref, deg_ref, ws_ref, wm_ref, b_ref):
    rep = jnp.concatenate([rep_ref[j] for j in range(NQ)], axis=1)
    agg = jnp.concatenate(
        [agg0_ref[0], agg0_ref[1], agg1_ref[0], agg1_ref[1]], axis=1)
    deg = jnp.maximum(jnp.sum(deg_ref[...], axis=0), 1.0)
    agg = agg / deg
    h = (jnp.dot(rep, ws_ref[...], preferred_element_type=jnp.float32)
         + jnp.dot(agg, wm_ref[...], preferred_element_type=jnp.float32)
         + b_ref[...])
    return jnp.maximum(h, 0.0)


def _update_mid_body(rep_ref, agg0_ref, agg1_ref, deg_ref, ws_ref, wm_ref,
                     b_ref, out_ref):
    h = _assemble(rep_ref, agg0_ref, agg1_ref, deg_ref, ws_ref, wm_ref, b_ref)
    for j in range(NQ):
        out_ref[j] = h[:, FQ * j:FQ * (j + 1)]


def _update_mid(rep4, agg0, agg1, degp3, W_upd_self, W_upd_msg, b_upd):
    grid = NA // BN
    return pl.pallas_call(
        _update_mid_body,
        grid=(grid,),
        in_specs=[
            pl.BlockSpec((NQ, BN, FQ), lambda i: (0, i, 0)),
            pl.BlockSpec((2, BN, FQ), lambda i: (0, i, 0)),
            pl.BlockSpec((2, BN, FQ), lambda i: (0, i, 0)),
            pl.BlockSpec((32, BN, 1), lambda i: (0, i, 0)),
            pl.BlockSpec((H, H), lambda i: (0, 0)),
            pl.BlockSpec((H, H), lambda i: (0, 0)),
            pl.BlockSpec((1, H), lambda i: (0, 0)),
        ],
        out_specs=pl.BlockSpec((NQ, BN, FQ), lambda i: (0, i, 0)),
        out_shape=jax.ShapeDtypeStruct((NQ, NA, FQ), jnp.float32),
    )(rep4, agg0, agg1, degp3, W_upd_self, W_upd_msg, b_upd.reshape(1, H))


def _update_final_body(rep_ref, agg0_ref, agg1_ref, deg_ref, ws_ref, wm_ref,
                       b_ref, wc_ref, bc_ref, out_ref):
    h = _assemble(rep_ref, agg0_ref, agg1_ref, deg_ref, ws_ref, wm_ref, b_ref)
    out_ref[...] = (jnp.dot(h, wc_ref[...], preferred_element_type=jnp.float32)
                    + bc_ref[...])


def _update_final(rep4, agg0, agg1, degp3, W_upd_self, W_upd_msg, b_upd,
                  W_cat, b_cat):
    grid = NA // BN
    nout = C + H_OUT
    return pl.pallas_call(
        _update_final_body,
        grid=(grid,),
        in_specs=[
            pl.BlockSpec((NQ, BN, FQ), lambda i: (0, i, 0)),
            pl.BlockSpec((2, BN, FQ), lambda i: (0, i, 0)),
            pl.BlockSpec((2, BN, FQ), lambda i: (0, i, 0)),
            pl.BlockSpec((32, BN, 1), lambda i: (0, i, 0)),
            pl.BlockSpec((H, H), lambda i: (0, 0)),
            pl.BlockSpec((H, H), lambda i: (0, 0)),
            pl.BlockSpec((1, H), lambda i: (0, 0)),
            pl.BlockSpec((H, nout), lambda i: (0, 0)),
            pl.BlockSpec((1, nout), lambda i: (0, 0)),
        ],
        out_specs=pl.BlockSpec((BN, nout), lambda i: (i, 0)),
        out_shape=jax.ShapeDtypeStruct((NA, nout), jnp.float32),
    )(rep4, agg0, agg1, degp3, W_upd_self, W_upd_msg, b_upd.reshape(1, H),
      W_cat, b_cat.reshape(1, nout))


def kernel(x, pos_feats, obj_labels, rel_pair_idx, freq_bias, obj_embed_w,
           W_pos1, b_pos1, W_pos2, b_pos2, W_enc, b_enc, W_gate, b_gate,
           W_upd_self, W_upd_msg, b_upd, W_out, b_out, W_h, b_h):
    x_pad = jnp.pad(x, ((0, NA - N), (0, 0)))
    labels_pad = jnp.pad(obj_labels, (0, NA - N))
    labels2 = labels_pad.reshape(NA, 1)
    fb_pad = jnp.pad(freq_bias, ((0, NPA - C * C), (0, 0)))
    src = rel_pair_idx[:, 0]
    dst = rel_pair_idx[:, 1]
    src_pad = jnp.pad(src, (0, EPAD - E))
    dst_pad = jnp.pad(dst, (0, EPAD - E), constant_values=N)
    src_w = src_pad.reshape(32, PW)
    dst_w = dst_pad.reshape(32, PW)
    src_t = src_pad.reshape(16, TCH, KCH)
    dst_t = dst_pad.reshape(16, TCH, KCH)

    pair_w, degp = _pair_deg_kernel(labels_pad, src_w, dst_w)
    pair_t = pair_w.reshape(16, TCH, KCH)
    degp3 = degp.reshape(32, NA, 1)

    pos = _pos_mlp(pos_feats, W_pos1, b_pos1, W_pos2, b_pos2)
    pos_pad = jnp.pad(pos, ((0, NA - N), (0, 0)))
    rep4 = _encoder(x_pad, labels2, pos_pad, W_enc, b_enc, obj_embed_w)
    gate4 = _gate_table(fb_pad, W_gate, b_gate).reshape(NQ * NPA, FQ)

    W_cat = jnp.concatenate([W_out, W_h], axis=1)
    b_cat = jnp.concatenate([b_out, b_h], axis=0)
    for layer in range(N_LAYERS):
        rep_flat = rep4.reshape(NQ * NA, FQ)
        agg0 = _message_kernels[0](rep_flat, gate4, src_t, pair_t, dst_t)
        agg1 = _message_kernels[1](rep_flat, gate4, src_t, pair_t, dst_t)
        agg0 = agg0.reshape(2, NA, FQ)
        agg1 = agg1.reshape(2, NA, FQ)
        if layer < N_LAYERS - 1:
            rep4 = _update_mid(rep4, agg0, agg1, degp3, W_upd_self,
                               W_upd_msg, b_upd)
        else:
            out = _update_final(rep4, agg0, agg1, degp3, W_upd_self,
                                W_upd_msg, b_upd, W_cat, b_cat)
    return out[:N]


# speedup vs baseline: 1.0474x; 1.0044x over previous
import functools

import jax
import jax.numpy as jnp
from jax import lax
from jax.experimental import pallas as pl
from jax.experimental.pallas import tpu as pltpu
from jax.experimental.pallas import tpu_sc as plsc

N = 10000
E = 160000
C = 151
R = 51
EMB = 128
D_IN = 512
H = 256
HH = 128
H_OUT = 16
N_LAYERS = 2

FQ = 64
NQ = 4
NA = 10240
NPA = 23040
EPAD = 163840
TCH = 128
KCH = 80
PW = EPAD // 32
PCH = PW // 16
BN = 1280

_mesh = plsc.VectorSubcoreMesh(core_axis_name="c", subcore_axis_name="s")


@functools.partial(
    pl.kernel,
    out_type=[
        jax.ShapeDtypeStruct((32, PW), jnp.int32),
        jax.ShapeDtypeStruct((32, NA), jnp.float32),
    ],
    mesh=_mesh,
    scratch_types=[
        pltpu.VMEM((NA,), jnp.int32),
        pltpu.VMEM((PW,), jnp.int32),
        pltpu.VMEM((PW,), jnp.int32),
        pltpu.VMEM((PW,), jnp.int32),
        pltpu.VMEM((NA,), jnp.float32),
    ],
    compiler_params=pltpu.CompilerParams(needs_layout_passes=False),
)
def _pair_deg_kernel(labels_hbm, src_hbm, dst_hbm, pair_out, deg_out,
                     labels_v, src_v, dst_v, pair_v, deg_v):
    c = lax.axis_index("c")
    s = lax.axis_index("s")
    w = s * 2 + c
    pltpu.sync_copy(labels_hbm, labels_v)
    pltpu.sync_copy(src_hbm.at[w], src_v)
    pltpu.sync_copy(dst_hbm.at[w], dst_v)
    zeros16 = jnp.zeros((16,), jnp.float32)

    def zero_body(i, _):
        deg_v[pl.ds(i * 16, 16)] = zeros16
        return 0

    lax.fori_loop(0, NA // 16, zero_body, 0)
    ones16 = jnp.ones((16,), jnp.float32)

    def body(j, _):
        sv = src_v[pl.ds(j * 16, 16)]
        dv = dst_v[pl.ds(j * 16, 16)]
        ls = plsc.load_gather(labels_v, [sv])
        ld = plsc.load_gather(labels_v, [dv])
        pair_v[pl.ds(j * 16, 16)] = ls * C + ld
        plsc.addupdate_scatter(deg_v, [dv], ones16)
        return 0

    lax.fori_loop(0, PCH, body, 0)
    pltpu.sync_copy(pair_v, pair_out.at[w])
    pltpu.sync_copy(deg_v, deg_out.at[w])


def _make_message_kernel(q):
    @functools.partial(
        pl.kernel,
        out_type=jax.ShapeDtypeStruct((2 * NA, FQ), jnp.float32),
        mesh=_mesh,
        scratch_types=[
            pltpu.VMEM((TCH, KCH), jnp.int32),
            pltpu.VMEM((TCH, KCH), jnp.int32),
            pltpu.VMEM((TCH, KCH), jnp.int32),
            pltpu.VMEM((4, KCH, FQ), jnp.float32),
            pltpu.VMEM((4, KCH, FQ), jnp.float32),
            pltpu.VMEM((KCH, FQ), jnp.float32),
            pltpu.VMEM_SHARED((NA, FQ), jnp.float32),
        ] + [pltpu.SemaphoreType.DMA] * 12,
        compiler_params=pltpu.CompilerParams(needs_layout_passes=False,
                                             use_tc_tiling_on_sc=False),
    )
    def _message_kernel(rep_hbm, gate_hbm, src_hbm, pair_hbm, dst_hbm,
                        agg_out, src_v, pair_v, dst_v, rep_b, gate_b, zero_b,
                        acc, *sems):
        c = lax.axis_index("c")
        s = lax.axis_index("s")
        pltpu.sync_copy(src_hbm.at[s], src_v)
        pltpu.sync_copy(pair_hbm.at[s], pair_v)
        pltpu.sync_copy(dst_hbm.at[s], dst_v)
        koff = 2 * q + c
        c_rep = koff * NA
        c_gate = koff * NPA

        def off_body(j, _):
            for i in range(KCH // 16):
                sl = pl.ds(i * 16, 16)
                src_v[j, sl] = src_v[j, sl] + c_rep
                pair_v[j, sl] = pair_v[j, sl] + c_gate
            return 0

        lax.fori_loop(0, TCH, off_body, 0)

        zeros16 = jnp.zeros((16,), jnp.float32)

        def zb_body(r, _):
            for l in range(FQ // 16):
                zero_b[r, pl.ds(l * 16, 16)] = zeros16
            return 0

        lax.fori_loop(0, KCH, zb_body, 0)

        rows_per_tile = NA // 16

        def zacc_body(k, _):
            pltpu.sync_copy(zero_b,
                            acc.at[pl.ds(s * rows_per_tile + k * KCH, KCH)])
            return 0

        lax.fori_loop(0, rows_per_tile // KCH, zacc_body, 0)
        plsc.subcore_barrier()

        semr = sems[0:4]
        semg = sems[4:8]
        sems_sc = sems[8:12]
        RPG = 8

        def issue_gather(j, b):
            pltpu.async_copy(rep_hbm.at[src_v.at[j]], rep_b.at[b], semr[b])
            pltpu.async_copy(gate_hbm.at[pair_v.at[j]], gate_b.at[b], semg[b])

        def wait_gather(j, b):
            pltpu.make_async_copy(rep_hbm.at[src_v.at[j]], rep_b.at[b],
                                  semr[b]).wait()
            pltpu.make_async_copy(gate_hbm.at[pair_v.at[j]], gate_b.at[b],
                                  semg[b]).wait()

        def issue_scatter(j, b):
            pltpu.async_copy(rep_b.at[b], acc.at[dst_v.at[j]], sems_sc[b],
                             add=True)

        def wait_scatter(j, b):
            pltpu.make_async_copy(rep_b.at[b], acc.at[dst_v.at[j]],
                                  sems_sc[b]).wait()

        def mul(b):
            def mul_body(g, _):
                for rr in range(RPG):
                    for l in range(FQ // 16):
                        sl = pl.ds(l * 16, 16)
                        r = g * RPG + rr
                        rep_b[b, r, sl] = rep_b[b, r, sl] * gate_b[b, r, sl]
                return 0

            lax.fori_loop(0, KCH // RPG, mul_body, 0)

        issue_gather(0, 0)
        issue_gather(1, 1)
        issue_gather(2, 2)
        wait_gather(0, 0)
        mul(0)
        issue_scatter(0, 0)
        issue_gather(3, 3)

        def edge_body(jj, _):
            for i in range(4):
                j = jj * 4 + i + 1
                b = (i + 1) % 4
                bp = (b + 3) % 4
                wait_scatter(j - 1, bp)
                issue_gather(j + 3, bp)
                wait_gather(j, b)
                mul(b)
                issue_scatter(j, b)
            return 0

        lax.fori_loop(0, (TCH - 4) // 4, edge_body, 0)
        for j in (TCH - 3, TCH - 2, TCH - 1):
            b = j % 4
            wait_gather(j, b)
            mul(b)
            issue_scatter(j, b)
        for j in (TCH - 4, TCH - 3, TCH - 2, TCH - 1):
            wait_scatter(j, j % 4)
        plsc.subcore_barrier()

        def out_body(k, _):
            sl = pl.ds(s * rows_per_tile + k * KCH, KCH)
            osl = pl.ds(c * NA + s * rows_per_tile + k * KCH, KCH)
            pltpu.sync_copy(acc.at[sl], agg_out.at[osl])
            return 0

        lax.fori_loop(0, rows_per_tile // KCH, out_body, 0)

    return _message_kernel


_message_kernels = [_make_message_kernel(0), _make_message_kernel(1)]


def _pos_mlp_body(pf_ref, w1_ref, b1_ref, w2_ref, b2_ref, out_ref):
    h1 = jnp.dot(pf_ref[...], w1_ref[...], preferred_element_type=jnp.float32)
    h1 = h1 + b1_ref[...]
    mu = jnp.mean(h1, axis=0, keepdims=True)
    var = jnp.mean((h1 - mu) * (h1 - mu), axis=0, keepdims=True)
    h1 = (h1 - mu) / jnp.sqrt(var + 1e-5)
    h2 = jnp.dot(h1, w2_ref[...], preferred_element_type=jnp.float32)
    out_ref[...] = jnp.maximum(h2 + b2_ref[...], 0.0)


def _pos_mlp(pos_feats, W_pos1, b_pos1, W_pos2, b_pos2):
    return pl.pallas_call(
        _pos_mlp_body,
        out_shape=jax.ShapeDtypeStruct((N, 128), jnp.float32),
    )(pos_feats, W_pos1, b_pos1.reshape(1, 32), W_pos2, b_pos2.reshape(1, 128))


def _encoder_body(x_ref, lab_ref, pos_ref, wx_ref, we_ref, wp_ref, b_ref,
                  emb_ref, out_ref):
    lab = lab_ref[...]
    iota = lax.broadcasted_iota(jnp.int32, (BN, C), 1)
    onehot = (lab == iota).astype(jnp.float32)
    emb = jnp.dot(onehot, emb_ref[...], preferred_element_type=jnp.float32)
    rep = (jnp.dot(x_ref[...], wx_ref[...], preferred_element_type=jnp.float32)
           + jnp.dot(emb, we_ref[...], preferred_element_type=jnp.float32)
           + jnp.dot(pos_ref[...], wp_ref[...], preferred_element_type=jnp.float32)
           + b_ref[...])
    for j in range(NQ):
        out_ref[j] = rep[:, FQ * j:FQ * (j + 1)]


def _encoder(x_pad, labels2, pos_pad, W_enc, b_enc, obj_embed_w):
    wx = W_enc[:D_IN]
    we = W_enc[D_IN:D_IN + EMB]
    wp = W_enc[D_IN + EMB:]
    grid = NA // BN
    return pl.pallas_call(
        _encoder_body,
        grid=(grid,),
        in_specs=[
            pl.BlockSpec((BN, D_IN), lambda i: (i, 0)),
            pl.BlockSpec((BN, 1), lambda i: (i, 0)),
            pl.BlockSpec((BN, 128), lambda i: (i, 0)),
            pl.BlockSpec((D_IN, H), lambda i: (0, 0)),
            pl.BlockSpec((EMB, H), lambda i: (0, 0)),
            pl.BlockSpec((128, H), lambda i: (0, 0)),
            pl.BlockSpec((1, H), lambda i: (0, 0)),
            pl.BlockSpec((C, EMB), lambda i: (0, 0)),
        ],
        out_specs=pl.BlockSpec((NQ, BN, FQ), lambda i: (0, i, 0)),
        out_shape=jax.ShapeDtypeStruct((NQ, NA, FQ), jnp.float32),
    )(x_pad, labels2, pos_pad, wx, we, wp, b_enc.reshape(1, H), obj_embed_w)


def _gate_body(fb_ref, w_ref, b_ref, out_ref):
    g = jnp.dot(fb_ref[...], w_ref[...], preferred_element_type=jnp.float32)
    g = jax.nn.sigmoid(g + b_ref[...])
    for j in range(NQ):
        out_ref[j] = g[:, FQ * j:FQ * (j + 1)]


def _gate_table(fb_pad, W_gate, b_gate):
    grid = NPA // BN
    return pl.pallas_call(
        _gate_body,
        grid=(grid,),
        in_specs=[
            pl.BlockSpec((BN, R), lambda i: (i, 0)),
            pl.BlockSpec((R, H), lambda i: (0, 0)),
            pl.BlockSpec((1, H), lambda i: (0, 0)),
        ],
        out_specs=pl.BlockSpec((NQ, BN, FQ), lambda i: (0, i, 0)),
        out_shape=jax.ShapeDtypeStruct((NQ, NPA, FQ), jnp.float32),
    )(fb_pad, W_gate, b_gate.reshape(1, H))


def _assemble(rep_ref, agg0_ref, agg1_ref, deg_ref, ws_ref, wm_ref, b_ref):
    rep = jnp.concatenate([rep_ref[j] for j in range(NQ)], axis=1)
    agg = jnp.concatenate(
        [agg0_ref[0], agg0_ref[1], agg1_ref[0], agg1_ref[1]], axis=1)
    deg = jnp.maximum(jnp.sum(deg_ref[...], axis=0), 1.0)
    agg = agg / deg
    h = (jnp.dot(rep, ws_ref[...], preferred_element_type=jnp.float32)
         + jnp.dot(agg, wm_ref[...], preferred_element_type=jnp.float32)
         + b_ref[...])
    return jnp.maximum(h, 0.0)


def _update_mid_body(rep_ref, agg0_ref, agg1_ref, deg_ref, ws_ref, wm_ref,
                     b_ref, out_ref):
    h = _assemble(rep_ref, agg0_ref, agg1_ref, deg_ref, ws_ref, wm_ref, b_ref)
    for j in range(NQ):
        out_ref[j] = h[:, FQ * j:FQ * (j + 1)]


def _update_mid(rep4, agg0, agg1, degp3, W_upd_self, W_upd_msg, b_upd):
    grid = NA // BN
    return pl.pallas_call(
        _update_mid_body,
        grid=(grid,),
        in_specs=[
            pl.BlockSpec((NQ, BN, FQ), lambda i: (0, i, 0)),
            pl.BlockSpec((2, BN, FQ), lambda i: (0, i, 0)),
            pl.BlockSpec((2, BN, FQ), lambda i: (0, i, 0)),
            pl.BlockSpec((32, BN, 1), lambda i: (0, i, 0)),
            pl.BlockSpec((H, H), lambda i: (0, 0)),
            pl.BlockSpec((H, H), lambda i: (0, 0)),
            pl.BlockSpec((1, H), lambda i: (0, 0)),
        ],
        out_specs=pl.BlockSpec((NQ, BN, FQ), lambda i: (0, i, 0)),
        out_shape=jax.ShapeDtypeStruct((NQ, NA, FQ), jnp.float32),
    )(rep4, agg0, agg1, degp3, W_upd_self, W_upd_msg, b_upd.reshape(1, H))


def _update_final_body(rep_ref, agg0_ref, agg1_ref, deg_ref, ws_ref, wm_ref,
                       b_ref, wc_ref, bc_ref, out_ref):
    h = _assemble(rep_ref, agg0_ref, agg1_ref, deg_ref, ws_ref, wm_ref, b_ref)
    out_ref[...] = (jnp.dot(h, wc_ref[...], preferred_element_type=jnp.float32)
                    + bc_ref[...])


def _update_final(rep4, agg0, agg1, degp3, W_upd_self, W_upd_msg, b_upd,
                  W_cat, b_cat):
    grid = NA // BN
    nout = C + H_OUT
    return pl.pallas_call(
        _update_final_body,
        grid=(grid,),
        in_specs=[
            pl.BlockSpec((NQ, BN, FQ), lambda i: (0, i, 0)),
            pl.BlockSpec((2, BN, FQ), lambda i: (0, i, 0)),
            pl.BlockSpec((2, BN, FQ), lambda i: (0, i, 0)),
            pl.BlockSpec((32, BN, 1), lambda i: (0, i, 0)),
            pl.BlockSpec((H, H), lambda i: (0, 0)),
            pl.BlockSpec((H, H), lambda i: (0, 0)),
            pl.BlockSpec((1, H), lambda i: (0, 0)),
            pl.BlockSpec((H, nout), lambda i: (0, 0)),
            pl.BlockSpec((1, nout), lambda i: (0, 0)),
        ],
        out_specs=pl.BlockSpec((BN, nout), lambda i: (i, 0)),
        out_shape=jax.ShapeDtypeStruct((NA, nout), jnp.float32),
    )(rep4, agg0, agg1, degp3, W_upd_self, W_upd_msg, b_upd.reshape(1, H),
      W_cat, b_cat.reshape(1, nout))


def kernel(x, pos_feats, obj_labels, rel_pair_idx, freq_bias, obj_embed_w,
           W_pos1, b_pos1, W_pos2, b_pos2, W_enc, b_enc, W_gate, b_gate,
           W_upd_self, W_upd_msg, b_upd, W_out, b_out, W_h, b_h):
    x_pad = jnp.pad(x, ((0, NA - N), (0, 0)))
    labels_pad = jnp.pad(obj_labels, (0, NA - N))
    labels2 = labels_pad.reshape(NA, 1)
    fb_pad = jnp.pad(freq_bias, ((0, NPA - C * C), (0, 0)))
    src = rel_pair_idx[:, 0]
    dst = rel_pair_idx[:, 1]
    src_pad = jnp.pad(src, (0, EPAD - E))
    dst_pad = jnp.pad(dst, (0, EPAD - E), constant_values=N)
    src_w = src_pad.reshape(32, PW)
    dst_w = dst_pad.reshape(32, PW)
    src_t = src_pad.reshape(16, TCH, KCH)
    dst_t = dst_pad.reshape(16, TCH, KCH)

    pair_w, degp = _pair_deg_kernel(labels_pad, src_w, dst_w)
    pair_t = pair_w.reshape(16, TCH, KCH)
    degp3 = degp.reshape(32, NA, 1)

    pos = _pos_mlp(pos_feats, W_pos1, b_pos1, W_pos2, b_pos2)
    pos_pad = jnp.pad(pos, ((0, NA - N), (0, 0)))
    rep4 = _encoder(x_pad, labels2, pos_pad, W_enc, b_enc, obj_embed_w)
    gate4 = _gate_table(fb_pad, W_gate, b_gate).reshape(NQ * NPA, FQ)

    W_cat = jnp.concatenate([W_out, W_h], axis=1)
    b_cat = jnp.concatenate([b_out, b_h], axis=0)
    for layer in range(N_LAYERS):
        rep_flat = rep4.reshape(NQ * NA, FQ)
        agg0 = _message_kernels[0](rep_flat, gate4, src_t, pair_t, dst_t)
        agg1 = _message_kernels[1](rep_flat, gate4, src_t, pair_t, dst_t)
        agg0 = agg0.reshape(2, NA, FQ)
        agg1 = agg1.reshape(2, NA, FQ)
        if layer < N_LAYERS - 1:
            rep4 = _update_mid(rep4, agg0, agg1, degp3, W_upd_self,
                               W_upd_msg, b_upd)
        else:
            out = _update_final(rep4, agg0, agg1, degp3, W_upd_self,
                                W_upd_msg, b_upd, W_cat, b_cat)
    return out[:N]
